# baseline (speedup 1.0000x reference)
import jax
import jax.numpy as jnp
from jax.experimental import pallas as pl
from jax.experimental.pallas import tpu as pltpu

_NH = 1024
_DT = 0.1
_DELAY = 512
_K = 0.1

_ROWS = 2048
_COLS = 32768
_NCHUNK = 8
_RC = _ROWS // _NCHUNK


def _body(ts_ref, buf_ref, w_ref, outb_ref, outnx_ref,
          ra_ref, rb_ref, rx_ref, csem, rsem, psem):
    ts = ts_ref[0]
    for k in range(_NCHUNK):
        pltpu.make_async_copy(
            buf_ref.at[pl.ds(k * _RC, _RC)],
            outb_ref.at[pl.ds(k * _RC, _RC)],
            csem.at[k],
        ).start()

    ba = ((_NH + ts - _DELAY) // 8) * 8
    bb = ((_NH + ts + 1 - _DELAY) // 8) * 8
    bx = ((_NH + ts) // 8) * 8
    ca = pltpu.make_async_copy(buf_ref.at[pl.ds(ba, 8)], ra_ref, rsem.at[0])
    cb = pltpu.make_async_copy(buf_ref.at[pl.ds(bb, 8)], rb_ref, rsem.at[1])
    cx = pltpu.make_async_copy(buf_ref.at[pl.ds(bx, 8)], rx_ref, rsem.at[2])
    ca.start()
    cb.start()
    cx.start()
    ca.wait()
    cb.wait()
    cx.wait()

    x = rx_ref[(_NH + ts) % 8, :]
    a = ra_ref[(_NH + ts - _DELAY) % 8, :]
    b = rb_ref[(_NH + ts + 1 - _DELAY) % 8, :]
    w = w_ref[...]
    d1 = -x + _K * jnp.tanh(a)
    xi = x + _DT * d1 + w
    d2 = -xi + _K * jnp.tanh(b)
    nx = x + _DT * 0.5 * (d1 + d2) + w
    outnx_ref[...] = nx

    for k in range(_NCHUNK):
        pltpu.make_async_copy(
            buf_ref.at[pl.ds(k * _RC, _RC)],
            outb_ref.at[pl.ds(k * _RC, _RC)],
            csem.at[k],
        ).wait()

    patch = pltpu.make_async_copy(outnx_ref, outb_ref.at[_NH + ts + 1], psem)
    patch.start()
    patch.wait()


def kernel(buf, dWt, t):
    ts = t[0, 0:1].astype(jnp.int32)
    grid_spec = pltpu.PrefetchScalarGridSpec(
        num_scalar_prefetch=1,
        grid=(1,),
        in_specs=[
            pl.BlockSpec(memory_space=pltpu.MemorySpace.HBM),
            pl.BlockSpec((_COLS,), lambda i, ts: (0,)),
        ],
        out_specs=[
            pl.BlockSpec(memory_space=pltpu.MemorySpace.HBM),
            pl.BlockSpec((_COLS,), lambda i, ts: (0,)),
        ],
        scratch_shapes=[
            pltpu.VMEM((8, _COLS), jnp.float32),
            pltpu.VMEM((8, _COLS), jnp.float32),
            pltpu.VMEM((8, _COLS), jnp.float32),
            pltpu.SemaphoreType.DMA((_NCHUNK,)),
            pltpu.SemaphoreType.DMA((3,)),
            pltpu.SemaphoreType.DMA,
        ],
    )
    buf2, nx = pl.pallas_call(
        _body,
        grid_spec=grid_spec,
        out_shape=[
            jax.ShapeDtypeStruct((_ROWS, _COLS), jnp.float32),
            jax.ShapeDtypeStruct((_COLS,), jnp.float32),
        ],
    )(ts, buf, dWt)
    return (buf2, nx)


# speedup vs baseline: 47.6315x; 47.6315x over previous
import jax
import jax.numpy as jnp
from jax.experimental import pallas as pl
from jax.experimental.pallas import tpu as pltpu

_NH = 1024
_DT = 0.1
_DELAY = 512
_K = 0.1

_ROWS = 2048
_COLS = 32768
_R = 32
_GRID = _ROWS // _R


def _body(ts_ref, buf_ref, rowa_ref, rowb_ref, rowx_ref, w_ref,
          outb_ref, outnx_ref, nx_ref):
    ts = ts_ref[0]
    i = pl.program_id(0)
    outb_ref[...] = buf_ref[...]

    @pl.when(i == 0)
    def _compute():
        x = rowx_ref[(_NH + ts) % 8, :]
        a = rowa_ref[(_NH + ts - _DELAY) % 8, :]
        b = rowb_ref[(_NH + ts + 1 - _DELAY) % 8, :]
        w = w_ref[...]
        d1 = -x + _K * jnp.tanh(a)
        xi = x + _DT * d1 + w
        d2 = -xi + _K * jnp.tanh(b)
        nx = x + _DT * 0.5 * (d1 + d2) + w
        outnx_ref[...] = nx
        nx_ref[...] = nx

    @pl.when(i == (_NH + ts + 1) // _R)
    def _patch():
        outb_ref[(_NH + ts + 1) % _R, :] = nx_ref[...]


def kernel(buf, dWt, t):
    ts = t[0, 0:1].astype(jnp.int32)
    grid_spec = pltpu.PrefetchScalarGridSpec(
        num_scalar_prefetch=1,
        grid=(_GRID,),
        in_specs=[
            pl.BlockSpec((_R, _COLS), lambda i, ts: (i, 0)),
            pl.BlockSpec((8, _COLS), lambda i, ts: ((_NH + ts[0] - _DELAY) // 8, 0)),
            pl.BlockSpec((8, _COLS), lambda i, ts: ((_NH + ts[0] + 1 - _DELAY) // 8, 0)),
            pl.BlockSpec((8, _COLS), lambda i, ts: ((_NH + ts[0]) // 8, 0)),
            pl.BlockSpec((_COLS,), lambda i, ts: (0,)),
        ],
        out_specs=[
            pl.BlockSpec((_R, _COLS), lambda i, ts: (i, 0)),
            pl.BlockSpec((_COLS,), lambda i, ts: (0,)),
        ],
        scratch_shapes=[pltpu.VMEM((_COLS,), jnp.float32)],
    )
    buf2, nx = pl.pallas_call(
        _body,
        grid_spec=grid_spec,
        out_shape=[
            jax.ShapeDtypeStruct((_ROWS, _COLS), jnp.float32),
            jax.ShapeDtypeStruct((_COLS,), jnp.float32),
        ],
        compiler_params=pltpu.CompilerParams(vmem_limit_bytes=112 * 1024 * 1024),
    )(ts, buf, buf, buf, buf, dWt)
    return (buf2, nx)


# speedup vs baseline: 47.6615x; 1.0006x over previous
import jax
import jax.numpy as jnp
from jax.experimental import pallas as pl
from jax.experimental.pallas import tpu as pltpu

_NH = 1024
_DT = 0.1
_DELAY = 512
_K = 0.1

_ROWS = 2048
_COLS = 32768
_RC = 64
_NCHUNK = _ROWS // _RC
_NSLOT = 6
_LOOKAHEAD = 3


def _body(ts_ref, buf_ref, w_ref, outb_ref, outnx_ref, *rest):
    slots = rest[:_NSLOT]
    ra_ref, rb_ref, rx_ref, nx_ref, in_sem, out_sem, rsem = rest[_NSLOT:]
    ts = ts_ref[0]

    def in_copy(s, k):
        return pltpu.make_async_copy(
            buf_ref.at[pl.ds(s * _RC, _RC)], slots[k], in_sem.at[k])

    def out_copy(s, k):
        return pltpu.make_async_copy(
            slots[k], outb_ref.at[pl.ds(s * _RC, _RC)], out_sem.at[k])

    ba = ((_NH + ts - _DELAY) // 8) * 8
    bb = ((_NH + ts + 1 - _DELAY) // 8) * 8
    bx = ((_NH + ts) // 8) * 8
    ca = pltpu.make_async_copy(buf_ref.at[pl.ds(ba, 8)], ra_ref, rsem.at[0])
    cb = pltpu.make_async_copy(buf_ref.at[pl.ds(bb, 8)], rb_ref, rsem.at[1])
    cx = pltpu.make_async_copy(buf_ref.at[pl.ds(bx, 8)], rx_ref, rsem.at[2])
    ca.start()
    cb.start()
    cx.start()

    for s in range(_LOOKAHEAD):
        in_copy(s, s % _NSLOT).start()

    ca.wait()
    cb.wait()
    cx.wait()
    x = rx_ref[(_NH + ts) % 8, :]
    a = ra_ref[(_NH + ts - _DELAY) % 8, :]
    b = rb_ref[(_NH + ts + 1 - _DELAY) % 8, :]
    w = w_ref[...]
    d1 = -x + _K * jnp.tanh(a)
    xi = x + _DT * d1 + w
    d2 = -xi + _K * jnp.tanh(b)
    nx = x + _DT * 0.5 * (d1 + d2) + w
    outnx_ref[...] = nx
    nx_ref[...] = nx

    p = _NH + ts + 1
    sp = p // _RC

    for s in range(_NCHUNK):
        k = s % _NSLOT
        in_copy(s, k).wait()

        @pl.when(s == sp)
        def _patch():
            slots[k][p % _RC, :] = nx_ref[...]

        out_copy(s, k).start()
        nxt = s + _LOOKAHEAD
        if nxt < _NCHUNK:
            kn = nxt % _NSLOT
            prev = nxt - _NSLOT
            if prev >= 0:
                out_copy(prev, kn).wait()
            in_copy(nxt, kn).start()

    for s in range(_NCHUNK - _NSLOT, _NCHUNK):
        if s >= 0:
            out_copy(s, s % _NSLOT).wait()


def kernel(buf, dWt, t):
    ts = t[0, 0:1].astype(jnp.int32)
    grid_spec = pltpu.PrefetchScalarGridSpec(
        num_scalar_prefetch=1,
        grid=(1,),
        in_specs=[
            pl.BlockSpec(memory_space=pltpu.MemorySpace.HBM),
            pl.BlockSpec((_COLS,), lambda i, ts: (0,)),
        ],
        out_specs=[
            pl.BlockSpec(memory_space=pltpu.MemorySpace.HBM),
            pl.BlockSpec((_COLS,), lambda i, ts: (0,)),
        ],
        scratch_shapes=(
            [pltpu.VMEM((_RC, _COLS), jnp.float32) for _ in range(_NSLOT)]
            + [
                pltpu.VMEM((8, _COLS), jnp.float32),
                pltpu.VMEM((8, _COLS), jnp.float32),
                pltpu.VMEM((8, _COLS), jnp.float32),
                pltpu.VMEM((_COLS,), jnp.float32),
                pltpu.SemaphoreType.DMA((_NSLOT,)),
                pltpu.SemaphoreType.DMA((_NSLOT,)),
                pltpu.SemaphoreType.DMA((3,)),
            ]
        ),
    )
    buf2, nx = pl.pallas_call(
        _body,
        grid_spec=grid_spec,
        out_shape=[
            jax.ShapeDtypeStruct((_ROWS, _COLS), jnp.float32),
            jax.ShapeDtypeStruct((_COLS,), jnp.float32),
        ],
    )(ts, buf, dWt)
    return (buf2, nx)
